# baseline (speedup 1.0000x reference)
import functools

import jax
import jax.numpy as jnp
from jax import lax
from jax.experimental import pallas as pl
from jax.experimental.pallas import tpu as pltpu
from jax.experimental.pallas import tpu_sc as plsc

BATCH = 16384
FIELDS = 26
EMBED_DIM = 16

NUM_CORES = 2
NUM_SUBCORES = 16
NW = NUM_CORES * NUM_SUBCORES

B = BATCH * FIELDS
B_PER_W = B // NW
CHUNK = 1664
NCHUNKS = B_PER_W // CHUNK


@functools.lru_cache(maxsize=None)
def _build_gather():
    mesh = plsc.VectorSubcoreMesh(core_axis_name="c", subcore_axis_name="s",
                                  num_cores=NUM_CORES)

    @functools.partial(
        pl.kernel,
        mesh=mesh,
        out_type=jax.ShapeDtypeStruct((B, EMBED_DIM), jnp.float32),
        scratch_types=[
            pltpu.VMEM((CHUNK,), jnp.int32),
            pltpu.VMEM((CHUNK, EMBED_DIM), jnp.float32),
            pltpu.SemaphoreType.DMA,
        ],
        compiler_params=pltpu.CompilerParams(use_tc_tiling_on_sc=False),
    )
    def gather_kernel(idx_hbm, table_hbm, out_hbm, idx_v, rows_v, sem):
        wid = lax.axis_index("s") * NUM_CORES + lax.axis_index("c")
        base = wid * B_PER_W
        for ci in range(NCHUNKS):
            off = base + ci * CHUNK
            pltpu.sync_copy(idx_hbm.at[pl.ds(off, CHUNK)], idx_v)
            pltpu.async_copy(table_hbm.at[idx_v], rows_v, sem).wait()
            pltpu.sync_copy(rows_v, out_hbm.at[pl.ds(off, CHUNK)])

    return gather_kernel


def kernel(inputs, table):
    idx = inputs.reshape(-1).astype(jnp.int32)
    out = _build_gather()(idx, table)
    return out.reshape(BATCH, FIELDS, EMBED_DIM)


# speedup vs baseline: 1.0112x; 1.0112x over previous
import functools

import jax
import jax.numpy as jnp
from jax import lax
from jax.experimental import pallas as pl
from jax.experimental.pallas import tpu as pltpu
from jax.experimental.pallas import tpu_sc as plsc

BATCH = 16384
FIELDS = 26
EMBED_DIM = 16

NUM_CORES = 2
NUM_SUBCORES = 16
NW = NUM_CORES * NUM_SUBCORES

B = BATCH * FIELDS
B_PER_W = B // NW
CHUNK = 1664
NCHUNKS = B_PER_W // CHUNK
NBUF = 3


@functools.lru_cache(maxsize=None)
def _build_gather():
    mesh = plsc.VectorSubcoreMesh(core_axis_name="c", subcore_axis_name="s",
                                  num_cores=NUM_CORES)

    @functools.partial(
        pl.kernel,
        mesh=mesh,
        out_type=jax.ShapeDtypeStruct((B, EMBED_DIM), jnp.float32),
        scratch_types=[
            pltpu.VMEM((B_PER_W,), jnp.int32),
            [pltpu.VMEM((CHUNK, EMBED_DIM), jnp.float32) for _ in range(NBUF)],
            [pltpu.SemaphoreType.DMA for _ in range(NBUF)],
            [pltpu.SemaphoreType.DMA for _ in range(NBUF)],
        ],
        compiler_params=pltpu.CompilerParams(use_tc_tiling_on_sc=False),
    )
    def gather_kernel(idx_hbm, table_hbm, out_hbm, idx_v, rows, gsems, ssems):
        wid = lax.axis_index("s") * NUM_CORES + lax.axis_index("c")
        base = wid * B_PER_W
        pltpu.sync_copy(idx_hbm.at[pl.ds(base, B_PER_W)], idx_v)

        gathers = [None] * NCHUNKS
        stores = [None] * NCHUNKS
        for ci in range(min(NBUF, NCHUNKS)):
            gathers[ci] = pltpu.async_copy(
                table_hbm.at[idx_v.at[pl.ds(ci * CHUNK, CHUNK)]],
                rows[ci % NBUF], gsems[ci % NBUF])
        for ci in range(NCHUNKS):
            b = ci % NBUF
            gathers[ci].wait()
            stores[ci] = pltpu.async_copy(
                rows[b], out_hbm.at[pl.ds(base + ci * CHUNK, CHUNK)], ssems[b])
            nxt = ci + NBUF
            if nxt < NCHUNKS:
                stores[ci].wait()
                gathers[nxt] = pltpu.async_copy(
                    table_hbm.at[idx_v.at[pl.ds(nxt * CHUNK, CHUNK)]],
                    rows[b], gsems[b])
        for ci in range(max(0, NCHUNKS - NBUF), NCHUNKS):
            stores[ci].wait()

    return gather_kernel


def kernel(inputs, table):
    idx = inputs.reshape(-1).astype(jnp.int32)
    out = _build_gather()(idx, table)
    return out.reshape(BATCH, FIELDS, EMBED_DIM)
